# baseline (speedup 1.0000x reference)
import jax
import jax.numpy as jnp
from jax import lax
from jax.experimental import pallas as pl
from jax.experimental.pallas import tpu as pltpu
from jax.experimental.pallas import tpu_sc as plsc

_W = 256
_NBUF = 2
_NW = 32


def kernel(token_ids, weight):
    b, t = token_ids.shape
    n = b * t
    nv, d = weight.shape
    idx = token_ids.reshape(n).astype(jnp.int32)
    n_per = n // _NW
    nchunk = n_per // _W
    assert n_per % _W == 0 and nchunk % _NBUF == 0

    mesh = plsc.VectorSubcoreMesh(core_axis_name="core",
                                  subcore_axis_name="subcore")

    scratch = ([pltpu.VMEM((n_per,), jnp.int32)]
               + [pltpu.VMEM((_W, d), jnp.float32) for _ in range(_NBUF)]
               + [pltpu.SemaphoreType.DMA for _ in range(2 * _NBUF)])

    @pl.kernel(out_type=jax.ShapeDtypeStruct((n, d), weight.dtype),
               mesh=mesh,
               compiler_params=pltpu.CompilerParams(
                   use_tc_tiling_on_sc=False, disable_bounds_checks=True),
               scratch_types=scratch)
    def gather_kernel(table_hbm, idx_hbm, out_flat, idx_v, *rest):
        bufs = rest[:_NBUF]
        gsem = rest[_NBUF:2 * _NBUF]
        wsem = rest[2 * _NBUF:]
        wid = lax.axis_index("subcore") * 2 + lax.axis_index("core")
        base = wid * n_per
        pltpu.sync_copy(idx_hbm.at[pl.ds(base, n_per)], idx_v)

        def start_gather(bi, chunk):
            pltpu.make_async_copy(
                table_hbm.at[idx_v.at[pl.ds(chunk * _W, _W)]],
                bufs[bi], gsem[bi]).start()

        def wait_gather(bi):
            pltpu.make_async_copy(
                table_hbm.at[idx_v.at[pl.ds(0, _W)]],
                bufs[bi], gsem[bi]).wait()

        def start_wb(bi, chunk):
            pltpu.make_async_copy(
                bufs[bi], out_flat.at[pl.ds(base + chunk * _W, _W)],
                wsem[bi]).start()

        def wait_wb(bi):
            pltpu.make_async_copy(
                bufs[bi], out_flat.at[pl.ds(base, _W)], wsem[bi]).wait()

        for bi in range(_NBUF):
            start_gather(bi, bi)

        @pl.loop(0, nchunk - _NBUF, step=_NBUF)
        def _(g):
            for bi in range(_NBUF):
                chunk = g + bi
                wait_gather(bi)
                start_wb(bi, chunk)
                wait_wb(bi)
                start_gather(bi, chunk + _NBUF)

        for bi in range(_NBUF):
            wait_gather(bi)
            start_wb(bi, nchunk - _NBUF + bi)
        for bi in range(_NBUF):
            wait_wb(bi)

    return gather_kernel(weight, idx).reshape(b, t, d)


# speedup vs baseline: 1.0009x; 1.0009x over previous
import jax
import jax.numpy as jnp
from jax import lax
from jax.experimental import pallas as pl
from jax.experimental.pallas import tpu as pltpu
from jax.experimental.pallas import tpu_sc as plsc

_NBUF = 4
_NW = 32


def kernel(token_ids, weight):
    b, t = token_ids.shape
    nv, d = weight.shape
    b_per = b // _NW
    assert b % _NW == 0 and b_per % _NBUF == 0

    mesh = plsc.VectorSubcoreMesh(core_axis_name="core",
                                  subcore_axis_name="subcore")

    scratch = ([pltpu.VMEM((b_per, t), jnp.int32)]
               + [pltpu.VMEM((t, d), jnp.float32) for _ in range(_NBUF)]
               + [pltpu.SemaphoreType.DMA for _ in range(2 * _NBUF)])

    @pl.kernel(out_type=jax.ShapeDtypeStruct((b, t, d), weight.dtype),
               mesh=mesh,
               compiler_params=pltpu.CompilerParams(
                   use_tc_tiling_on_sc=False, disable_bounds_checks=True),
               scratch_types=scratch)
    def gather_kernel(table_hbm, idx_hbm, out_hbm, idx_v, *rest):
        bufs = rest[:_NBUF]
        gsem = rest[_NBUF:2 * _NBUF]
        wsem = rest[2 * _NBUF:]
        wid = lax.axis_index("subcore") * 2 + lax.axis_index("core")
        base = wid * b_per
        pltpu.sync_copy(idx_hbm.at[pl.ds(base, b_per)], idx_v)

        def start_gather(bi, j):
            pltpu.make_async_copy(
                table_hbm.at[idx_v.at[j]], bufs[bi], gsem[bi]).start()

        def wait_gather(bi):
            pltpu.make_async_copy(
                table_hbm.at[idx_v.at[0]], bufs[bi], gsem[bi]).wait()

        def start_wb(bi, j):
            pltpu.make_async_copy(
                bufs[bi], out_hbm.at[base + j], wsem[bi]).start()

        def wait_wb(bi):
            pltpu.make_async_copy(
                bufs[bi], out_hbm.at[base], wsem[bi]).wait()

        for bi in range(_NBUF):
            start_gather(bi, bi)

        @pl.loop(0, b_per - _NBUF, step=_NBUF)
        def _(g):
            for bi in range(_NBUF):
                j = g + bi
                wait_gather(bi)
                start_wb(bi, j)
                wait_wb(bi)
                start_gather(bi, j + _NBUF)

        for bi in range(_NBUF):
            wait_gather(bi)
            start_wb(bi, b_per - _NBUF + bi)
        for bi in range(_NBUF):
            wait_wb(bi)

    return gather_kernel(weight, token_ids.astype(jnp.int32))
